# Initial kernel scaffold; baseline (speedup 1.0000x reference)
import jax
import jax.numpy as jnp
from jax.experimental import pallas as pl


def kernel(x, weight, lora_a, lora_b):
    raise NotImplementedError("write your pallas kernel here")


# speedup vs baseline: 3.3645x; 3.3645x over previous
import functools

import jax
import jax.numpy as jnp
from jax import lax
from jax.experimental import pallas as pl
from jax.experimental.pallas import tpu as pltpu
from jax.experimental.pallas import tpu_sc as plsc

_D = 64
_RANK = 16
_SCALING = 16.0 / 16.0

_NC = 2
_NS = 16
_NW = _NC * _NS

_TOK = 4096 * 50
_BPW = _TOK // _NW
_CH = 640
_KROWS = _CH // 128
_NCHUNK = _BPW // _CH


def _sc_gather_build():
    mesh = plsc.VectorSubcoreMesh(core_axis_name="c", subcore_axis_name="s")

    @functools.partial(
        pl.kernel,
        out_type=(
            jax.ShapeDtypeStruct((_TOK, _D), jnp.float32),
            jax.ShapeDtypeStruct((_TOK, _RANK), jnp.float32),
        ),
        mesh=mesh,
        scratch_types=[
            pltpu.VMEM((_CH,), jnp.int32),
            pltpu.VMEM((_CH, _D), jnp.float32),
            pltpu.VMEM((_CH, _RANK), jnp.float32),
            pltpu.SemaphoreType.DMA,
        ],
        compiler_params=pltpu.CompilerParams(use_tc_tiling_on_sc=False),
    )
    def sc_gather(idx_hbm, w_hbm, at_hbm, g_out, a_out, idx_v, g_v, a_v, sem):
        wid = lax.axis_index("s") * _NC + lax.axis_index("c")
        tok_base = wid * _BPW

        @pl.loop(0, _NCHUNK)
        def _chunk(i):
            off = tok_base + i * _CH
            pltpu.sync_copy(idx_hbm.at[pl.ds(off, _CH)], idx_v)
            copies = []
            for j in range(_KROWS):
                ids = idx_v.at[pl.ds(j * 128, 128)]
                copies.append(
                    pltpu.async_copy(
                        w_hbm.at[ids], g_v.at[pl.ds(j * 128, 128)], sem
                    )
                )
                copies.append(
                    pltpu.async_copy(
                        at_hbm.at[ids], a_v.at[pl.ds(j * 128, 128)], sem
                    )
                )
            for c in copies:
                c.wait()
            pltpu.sync_copy(g_v, g_out.at[pl.ds(off, _CH)])
            pltpu.sync_copy(a_v, a_out.at[pl.ds(off, _CH)])

    return sc_gather


_sc_gather = _sc_gather_build()


def _combine_body(g_ref, a_ref, b_ref, o_ref):
    o_ref[...] = g_ref[...] + jnp.dot(
        a_ref[...], b_ref[...], preferred_element_type=jnp.float32
    )


def _tc_combine(g, a, bs):
    bt = 2048
    return pl.pallas_call(
        _combine_body,
        grid=(_TOK // bt,),
        in_specs=[
            pl.BlockSpec((bt, _D), lambda i: (i, 0)),
            pl.BlockSpec((bt, _RANK), lambda i: (i, 0)),
            pl.BlockSpec((_RANK, _D), lambda i: (0, 0)),
        ],
        out_specs=pl.BlockSpec((bt, _D), lambda i: (i, 0)),
        out_shape=jax.ShapeDtypeStruct((_TOK, _D), jnp.float32),
    )(g, a, bs)


@jax.jit
def kernel(x, weight, lora_a, lora_b):
    b, l = x.shape
    idx = x.reshape(_TOK).astype(jnp.int32)
    at = jnp.asarray(lora_a.T)
    bs = (lora_b * _SCALING).T
    g, a = _sc_gather(idx, weight, at)
    out = _tc_combine(g, a, bs)
    return out.reshape(b, l, _D)
